# Initial kernel scaffold; baseline (speedup 1.0000x reference)
import jax
import jax.numpy as jnp
from jax.experimental import pallas as pl


def kernel(node_states, edge_src_0, edge_dst_0, edge_src_1, edge_dst_1, node_to_graph_idx, ref_root_ids, ref_root_graph_idx, edge_feat_0, edge_feat_1, W0, W1, W_self, b):
    raise NotImplementedError("write your pallas kernel here")


# speedup vs baseline: 2.4194x; 2.4194x over previous
import jax
import jax.numpy as jnp
from jax import lax
from jax.experimental import pallas as pl
from jax.experimental.pallas import tpu as pltpu
from jax.experimental.pallas import tpu_sc as plsc

N = 10000
D = 128
H = 16
E = 160000

NS = 16
EPT = E // NS
CH = 80
NCHUNK = EPT // CH
N_PAD = 10240
RPT = N_PAD // NS
RCH = CH
NRCH = RPT // RCH

_ZVEC = 16


def _sc_body(y0_ref, y1_ref, z0_ref, z1_ref,
             src0_ref, dst0_ref, src1_ref, dst1_ref,
             acc_out_ref,
             src_idx, dst_idx, rows, zrows,
             acc_sh, sem):
    c = lax.axis_index("c")
    s = lax.axis_index("s")

    zero = jnp.zeros((_ZVEC,), jnp.float32)

    def zfill(i, _):
        for j in range(D // _ZVEC):
            rows[i, pl.ds(j * _ZVEC, _ZVEC)] = zero
        return 0

    lax.fori_loop(0, RCH, zfill, 0)
    for k in range(NRCH):
        r0 = s * RPT + k * RCH
        pltpu.sync_copy(rows, acc_sh.at[pl.ds(r0, RCH)])
    plsc.subcore_barrier()

    def process(y_ref, z_ref, src_ref, dst_ref):
        def chunk(i, _):
            off = s * EPT + i * CH
            pltpu.sync_copy(src_ref.at[pl.ds(off, CH)], src_idx)
            pltpu.sync_copy(dst_ref.at[pl.ds(off, CH)], dst_idx)
            pltpu.sync_copy(z_ref.at[pl.ds(off, CH)], zrows)
            pltpu.async_copy(y_ref.at[src_idx], rows, sem).wait()
            pltpu.sync_copy(rows, acc_sh.at[dst_idx], add=True)
            pltpu.sync_copy(zrows, acc_sh.at[dst_idx], add=True)
            return 0

        lax.fori_loop(0, NCHUNK, chunk, 0)

    @pl.when(c == 0)
    def _():
        process(y0_ref, z0_ref, src0_ref, dst0_ref)

    @pl.when(c == 1)
    def _():
        process(y1_ref, z1_ref, src1_ref, dst1_ref)

    plsc.subcore_barrier()

    for k in range(NRCH):
        r0 = s * RPT + k * RCH
        pltpu.sync_copy(acc_sh.at[pl.ds(r0, RCH)], rows)
        pltpu.sync_copy(rows, acc_out_ref.at[c, pl.ds(r0, RCH)])


_sc_aggregate = pl.kernel(
    _sc_body,
    out_type=jax.ShapeDtypeStruct((2, N_PAD, D), jnp.float32),
    mesh=plsc.VectorSubcoreMesh(core_axis_name="c", subcore_axis_name="s"),
    scratch_types=[
        pltpu.VMEM((CH,), jnp.int32),
        pltpu.VMEM((CH,), jnp.int32),
        pltpu.VMEM((CH, D), jnp.float32),
        pltpu.VMEM((CH, D), jnp.float32),
        pltpu.VMEM_SHARED((N_PAD, D), jnp.float32),
        pltpu.SemaphoreType.DMA,
    ],
)


YBLK = 1000


def _tc_y_body(ns_ref, w0a_ref, w1a_ref, y0_ref, y1_ref):
    x = ns_ref[...]
    y0_ref[...] = jnp.dot(x, w0a_ref[...], preferred_element_type=jnp.float32)
    y1_ref[...] = jnp.dot(x, w1a_ref[...], preferred_element_type=jnp.float32)


_tc_y = pl.pallas_call(
    _tc_y_body,
    grid=(N // YBLK,),
    in_specs=[
        pl.BlockSpec((YBLK, D), lambda i: (i, 0)),
        pl.BlockSpec((D, D), lambda i: (0, 0)),
        pl.BlockSpec((D, D), lambda i: (0, 0)),
    ],
    out_specs=[
        pl.BlockSpec((YBLK, D), lambda i: (i, 0)),
        pl.BlockSpec((YBLK, D), lambda i: (i, 0)),
    ],
    out_shape=[
        jax.ShapeDtypeStruct((N, D), jnp.float32),
        jax.ShapeDtypeStruct((N, D), jnp.float32),
    ],
)


ZBLK = 2000


def _tc_z_body(f0_ref, f1_ref, w0b_ref, w1b_ref, z0_ref, z1_ref):
    z0_ref[...] = jnp.dot(f0_ref[...], w0b_ref[...],
                          preferred_element_type=jnp.float32)
    z1_ref[...] = jnp.dot(f1_ref[...], w1b_ref[...],
                          preferred_element_type=jnp.float32)


_tc_z = pl.pallas_call(
    _tc_z_body,
    grid=(E // ZBLK,),
    in_specs=[
        pl.BlockSpec((ZBLK, H), lambda i: (i, 0)),
        pl.BlockSpec((ZBLK, H), lambda i: (i, 0)),
        pl.BlockSpec((H, D), lambda i: (0, 0)),
        pl.BlockSpec((H, D), lambda i: (0, 0)),
    ],
    out_specs=[
        pl.BlockSpec((ZBLK, D), lambda i: (i, 0)),
        pl.BlockSpec((ZBLK, D), lambda i: (i, 0)),
    ],
    out_shape=[
        jax.ShapeDtypeStruct((E, D), jnp.float32),
        jax.ShapeDtypeStruct((E, D), jnp.float32),
    ],
)


BLK = 1000


def _tc_post_body(ns_ref, acc_ref, ws_ref, b_ref, out_ref):
    o = jnp.dot(ns_ref[...], ws_ref[...], preferred_element_type=jnp.float32)
    o += acc_ref[0] + acc_ref[1] + b_ref[...]
    out_ref[...] = jnp.maximum(o, 0.0)


_tc_post = pl.pallas_call(
    _tc_post_body,
    grid=(N // BLK,),
    in_specs=[
        pl.BlockSpec((BLK, D), lambda i: (i, 0)),
        pl.BlockSpec((2, BLK, D), lambda i: (0, i, 0)),
        pl.BlockSpec((D, D), lambda i: (0, 0)),
        pl.BlockSpec((1, D), lambda i: (0, 0)),
    ],
    out_specs=pl.BlockSpec((BLK, D), lambda i: (i, 0)),
    out_shape=jax.ShapeDtypeStruct((N, D), jnp.float32),
)


def kernel(node_states, edge_src_0, edge_dst_0, edge_src_1, edge_dst_1,
           node_to_graph_idx, ref_root_ids, ref_root_graph_idx,
           edge_feat_0, edge_feat_1, W0, W1, W_self, b):
    del node_to_graph_idx, ref_root_ids, ref_root_graph_idx
    y0, y1 = _tc_y(node_states, W0[:D], W1[:D])
    z0, z1 = _tc_z(edge_feat_0, edge_feat_1, W0[D:], W1[D:])
    acc = _sc_aggregate(
        y0, y1, z0, z1,
        edge_src_0.astype(jnp.int32), edge_dst_0.astype(jnp.int32),
        edge_src_1.astype(jnp.int32), edge_dst_1.astype(jnp.int32),
    )
    return _tc_post(node_states, acc, W_self, b.reshape(1, D))


# speedup vs baseline: 3.7645x; 1.5560x over previous
import jax
import jax.numpy as jnp
from jax import lax
from jax.experimental import pallas as pl
from jax.experimental.pallas import tpu as pltpu
from jax.experimental.pallas import tpu_sc as plsc

N = 10000
D = 128
H = 16
E = 160000

NS = 16
EPT = E // NS
CH = 80
NCHUNK = EPT // CH
N_PAD = 10240
RPT = N_PAD // NS
RCH = CH
NRCH = RPT // RCH

_ZVEC = 16


def _sc_body(y0_ref, y1_ref, z0_ref, z1_ref,
             src0_ref, dst0_ref, src1_ref, dst1_ref,
             acc_out_ref,
             si0, si1, di0, di1, ro0, ro1, zr0, zr1,
             acc_sh,
             m_si0, m_si1, m_di0, m_di1, m_z0, m_z1,
             m_g0, m_g1, m_y0, m_y1, m_zs0, m_zs1):
    c = lax.axis_index("c")
    s = lax.axis_index("s")
    SI, DI, RO, ZR = (si0, si1), (di0, di1), (ro0, ro1), (zr0, zr1)
    M_SI, M_DI, M_Z = (m_si0, m_si1), (m_di0, m_di1), (m_z0, m_z1)
    M_G, M_Y, M_ZS = (m_g0, m_g1), (m_y0, m_y1), (m_zs0, m_zs1)

    zero = jnp.zeros((_ZVEC,), jnp.float32)

    def zfill(i, _):
        for j in range(D // _ZVEC):
            ro0[i, pl.ds(j * _ZVEC, _ZVEC)] = zero
        return 0

    lax.fori_loop(0, RCH, zfill, 0)
    for k in range(NRCH):
        r0 = s * RPT + k * RCH
        pltpu.sync_copy(ro0, acc_sh.at[pl.ds(r0, RCH)])
    plsc.subcore_barrier()

    def process(y_ref, z_ref, src_ref, dst_ref):
        base = s * EPT

        def issue_loads(j, b):
            off = base + j * CH
            pltpu.async_copy(src_ref.at[pl.ds(off, CH)], SI[b], M_SI[b])
            pltpu.async_copy(dst_ref.at[pl.ds(off, CH)], DI[b], M_DI[b])
            pltpu.async_copy(z_ref.at[pl.ds(off, CH)], ZR[b], M_Z[b])

        def wait_scats(b):
            pltpu.make_async_copy(RO[b], acc_sh.at[pl.ds(0, CH)], M_Y[b]).wait()
            pltpu.make_async_copy(ZR[b], acc_sh.at[pl.ds(0, CH)], M_ZS[b]).wait()

        def step(j, b, first):
            pltpu.make_async_copy(src_ref.at[pl.ds(0, CH)], SI[b], M_SI[b]).wait()
            pltpu.make_async_copy(dst_ref.at[pl.ds(0, CH)], DI[b], M_DI[b]).wait()
            pltpu.make_async_copy(z_ref.at[pl.ds(0, CH)], ZR[b], M_Z[b]).wait()
            pltpu.async_copy(y_ref.at[SI[b]], RO[b], M_G[b])
            if not first:
                wait_scats(1 - b)

            @pl.when(j + 1 < NCHUNK)
            def _():
                issue_loads(j + 1, 1 - b)

            pltpu.make_async_copy(y_ref.at[SI[b]], RO[b], M_G[b]).wait()
            pltpu.async_copy(RO[b], acc_sh.at[DI[b]], M_Y[b], add=True)
            pltpu.async_copy(ZR[b], acc_sh.at[DI[b]], M_ZS[b], add=True)

        issue_loads(0, 0)
        step(0, 0, True)

        def pair(t, _):
            step(2 * t + 1, 1, False)
            step(2 * t + 2, 0, False)
            return 0

        lax.fori_loop(0, (NCHUNK - 1) // 2, pair, 0)
        wait_scats(0)

    @pl.when(c == 0)
    def _():
        process(y0_ref, z0_ref, src0_ref, dst0_ref)

    @pl.when(c == 1)
    def _():
        process(y1_ref, z1_ref, src1_ref, dst1_ref)

    plsc.subcore_barrier()

    for k in range(NRCH):
        r0 = s * RPT + k * RCH
        pltpu.sync_copy(acc_sh.at[pl.ds(r0, RCH)], ro0)
        pltpu.sync_copy(ro0, acc_out_ref.at[c, pl.ds(r0, RCH)])


_sc_aggregate = pl.kernel(
    _sc_body,
    out_type=jax.ShapeDtypeStruct((2, N_PAD, D), jnp.float32),
    mesh=plsc.VectorSubcoreMesh(core_axis_name="c", subcore_axis_name="s"),
    scratch_types=[
        pltpu.VMEM((CH,), jnp.int32),
        pltpu.VMEM((CH,), jnp.int32),
        pltpu.VMEM((CH,), jnp.int32),
        pltpu.VMEM((CH,), jnp.int32),
        pltpu.VMEM((CH, D), jnp.float32),
        pltpu.VMEM((CH, D), jnp.float32),
        pltpu.VMEM((CH, D), jnp.float32),
        pltpu.VMEM((CH, D), jnp.float32),
        pltpu.VMEM_SHARED((N_PAD, D), jnp.float32),
        pltpu.SemaphoreType.DMA,
        pltpu.SemaphoreType.DMA,
        pltpu.SemaphoreType.DMA,
        pltpu.SemaphoreType.DMA,
        pltpu.SemaphoreType.DMA,
        pltpu.SemaphoreType.DMA,
        pltpu.SemaphoreType.DMA,
        pltpu.SemaphoreType.DMA,
        pltpu.SemaphoreType.DMA,
        pltpu.SemaphoreType.DMA,
        pltpu.SemaphoreType.DMA,
        pltpu.SemaphoreType.DMA,
    ],
)


YBLK = 1000


def _tc_y_body(ns_ref, w0a_ref, w1a_ref, y0_ref, y1_ref):
    x = ns_ref[...]
    y0_ref[...] = jnp.dot(x, w0a_ref[...], preferred_element_type=jnp.float32)
    y1_ref[...] = jnp.dot(x, w1a_ref[...], preferred_element_type=jnp.float32)


_tc_y = pl.pallas_call(
    _tc_y_body,
    grid=(N // YBLK,),
    in_specs=[
        pl.BlockSpec((YBLK, D), lambda i: (i, 0)),
        pl.BlockSpec((D, D), lambda i: (0, 0)),
        pl.BlockSpec((D, D), lambda i: (0, 0)),
    ],
    out_specs=[
        pl.BlockSpec((YBLK, D), lambda i: (i, 0)),
        pl.BlockSpec((YBLK, D), lambda i: (i, 0)),
    ],
    out_shape=[
        jax.ShapeDtypeStruct((N, D), jnp.float32),
        jax.ShapeDtypeStruct((N, D), jnp.float32),
    ],
)


ZBLK = 2000


def _tc_z_body(f0_ref, f1_ref, w0b_ref, w1b_ref, z0_ref, z1_ref):
    z0_ref[...] = jnp.dot(f0_ref[...], w0b_ref[...],
                          preferred_element_type=jnp.float32)
    z1_ref[...] = jnp.dot(f1_ref[...], w1b_ref[...],
                          preferred_element_type=jnp.float32)


_tc_z = pl.pallas_call(
    _tc_z_body,
    grid=(E // ZBLK,),
    in_specs=[
        pl.BlockSpec((ZBLK, H), lambda i: (i, 0)),
        pl.BlockSpec((ZBLK, H), lambda i: (i, 0)),
        pl.BlockSpec((H, D), lambda i: (0, 0)),
        pl.BlockSpec((H, D), lambda i: (0, 0)),
    ],
    out_specs=[
        pl.BlockSpec((ZBLK, D), lambda i: (i, 0)),
        pl.BlockSpec((ZBLK, D), lambda i: (i, 0)),
    ],
    out_shape=[
        jax.ShapeDtypeStruct((E, D), jnp.float32),
        jax.ShapeDtypeStruct((E, D), jnp.float32),
    ],
)


BLK = 1000


def _tc_post_body(ns_ref, acc_ref, ws_ref, b_ref, out_ref):
    o = jnp.dot(ns_ref[...], ws_ref[...], preferred_element_type=jnp.float32)
    o += acc_ref[0] + acc_ref[1] + b_ref[...]
    out_ref[...] = jnp.maximum(o, 0.0)


_tc_post = pl.pallas_call(
    _tc_post_body,
    grid=(N // BLK,),
    in_specs=[
        pl.BlockSpec((BLK, D), lambda i: (i, 0)),
        pl.BlockSpec((2, BLK, D), lambda i: (0, i, 0)),
        pl.BlockSpec((D, D), lambda i: (0, 0)),
        pl.BlockSpec((1, D), lambda i: (0, 0)),
    ],
    out_specs=pl.BlockSpec((BLK, D), lambda i: (i, 0)),
    out_shape=jax.ShapeDtypeStruct((N, D), jnp.float32),
)


def kernel(node_states, edge_src_0, edge_dst_0, edge_src_1, edge_dst_1,
           node_to_graph_idx, ref_root_ids, ref_root_graph_idx,
           edge_feat_0, edge_feat_1, W0, W1, W_self, b):
    del node_to_graph_idx, ref_root_ids, ref_root_graph_idx
    y0, y1 = _tc_y(node_states, W0[:D], W1[:D])
    z0, z1 = _tc_z(edge_feat_0, edge_feat_1, W0[D:], W1[D:])
    acc = _sc_aggregate(
        y0, y1, z0, z1,
        edge_src_0.astype(jnp.int32), edge_dst_0.astype(jnp.int32),
        edge_src_1.astype(jnp.int32), edge_dst_1.astype(jnp.int32),
    )
    return _tc_post(node_states, acc, W_self, b.reshape(1, D))
